# baseline (speedup 1.0000x reference)
import jax
import jax.numpy as jnp
from jax import lax
from jax.experimental import pallas as pl
from jax.experimental.pallas import tpu as pltpu


def _push_tile_pair(w, mxu):
    H = w.shape[0]
    zfill = jnp.zeros((H, 2 * H), jnp.bfloat16)
    wb = w.astype(jnp.bfloat16)
    pltpu.matmul_push_rhs(
        jnp.concatenate([wb[:, 0:2 * H], zfill], axis=0), 0, mxu)
    pltpu.matmul_push_rhs(
        jnp.concatenate([wb[:, 2 * H:4 * H], zfill], axis=0), 1, mxu)


def _bilstm_kernel(x_ref, lens_ref, wihf_ref, bf_ref, whhf_ref,
                   wihb_ref, bb_ref, whhb_ref, w1_ref, b1_ref, w2_ref, b2_ref,
                   out_ref, xgf_ref, xgb_ref):
    T, BC, E = x_ref.shape
    H = whhf_ref.shape[0]
    TB = T * BC

    x2 = x_ref[...].reshape(TB, E)
    x2 = x2.astype(jnp.bfloat16)
    x_aug = jnp.concatenate([x2, x2], axis=1)
    bgf = bf_ref[...]
    bgb = bb_ref[...]
    _push_tile_pair(wihf_ref[...], 0)
    _push_tile_pair(wihb_ref[...], 1)
    CH = 256
    NCHUNK = TB // CH

    def _proj_issue(k):
        r = k * CH
        a = 128 * (k % 2)
        xa = x_aug[r:r + CH, :]
        pltpu.matmul_acc_lhs(a, xa, 0, load_staged_rhs=0)
        pltpu.matmul_acc_lhs(a, xa, 1, load_staged_rhs=0)
        pltpu.matmul_acc_lhs(a + 64, xa, 0, load_staged_rhs=1)
        pltpu.matmul_acc_lhs(a + 64, xa, 1, load_staged_rhs=1)

    def _proj_pop(k):
        r = k * CH
        a = 128 * (k % 2)
        f0 = pltpu.matmul_pop(a, (CH, 2 * H), jnp.float32, 0)
        b0 = pltpu.matmul_pop(a, (CH, 2 * H), jnp.float32, 1)
        f1 = pltpu.matmul_pop(a + 64, (CH, 2 * H), jnp.float32, 0)
        b1v = pltpu.matmul_pop(a + 64, (CH, 2 * H), jnp.float32, 1)
        xgf_ref[pl.ds(r, CH), :] = jnp.concatenate([f0, f1], axis=1) + bgf
        xgb_ref[pl.ds(r, CH), :] = jnp.concatenate([b0, b1v], axis=1) + bgb

    _proj_issue(0)
    for k in range(1, NCHUNK):
        _proj_issue(k)
        _proj_pop(k - 1)
    _proj_pop(NCHUNK - 1)

    zrow = jnp.zeros((H, 2 * H), jnp.bfloat16)
    whh_f = whhf_ref[...].astype(jnp.bfloat16)
    whh_b = whhb_ref[...].astype(jnp.bfloat16)
    pltpu.matmul_push_rhs(
        jnp.concatenate([whh_f[:, 0:2 * H], zrow], axis=0), 0, 0)
    pltpu.matmul_push_rhs(
        jnp.concatenate([whh_f[:, 2 * H:4 * H], zrow], axis=0), 1, 0)
    pltpu.matmul_push_rhs(
        jnp.concatenate([whh_b[:, 0:2 * H], zrow], axis=0), 0, 1)
    pltpu.matmul_push_rhs(
        jnp.concatenate([whh_b[:, 2 * H:4 * H], zrow], axis=0), 1, 1)

    len_h = jnp.broadcast_to(lens_ref[...], (BC, H))
    cap_f_th = len_h - 1
    th_b = T - len_h

    zeros = jnp.zeros((BC, H), jnp.float32)

    def gates(g, c):
        sig_if = 0.5 * jnp.tanh(0.5 * g[:, 0:2 * H]) + 0.5
        g_c = jnp.tanh(g[:, 2 * H:3 * H])
        sig_o = 0.5 * jnp.tanh(0.5 * g[:, 3 * H:4 * H]) + 0.5
        c_new = sig_if[:, H:2 * H] * c + sig_if[:, 0:H] * g_c
        h_new = sig_o * jnp.tanh(c_new)
        return h_new, c_new

    def body(s, carry):
        h_f, c_f, h_b, c_b, out_f, out_b = carry
        rf = pl.multiple_of(s * BC, BC)
        rb = pl.multiple_of((T - 1 - s) * BC, BC)
        hb_f = h_f.astype(jnp.bfloat16)
        hb_b = h_b.astype(jnp.bfloat16)
        hp_f = jnp.concatenate([hb_f, hb_f], axis=1)
        hp_b = jnp.concatenate([hb_b, hb_b], axis=1)
        pltpu.matmul_acc_lhs(0, hp_f, 0, load_staged_rhs=0)
        pltpu.matmul_acc_lhs(64, hp_f, 0, load_staged_rhs=1)
        pltpu.matmul_acc_lhs(0, hp_b, 1, load_staged_rhs=0)
        pltpu.matmul_acc_lhs(64, hp_b, 1, load_staged_rhs=1)
        mf0 = pltpu.matmul_pop(0, (BC, 2 * H), jnp.float32, 0)
        mf1 = pltpu.matmul_pop(64, (BC, 2 * H), jnp.float32, 0)
        mb0 = pltpu.matmul_pop(0, (BC, 2 * H), jnp.float32, 1)
        mb1 = pltpu.matmul_pop(64, (BC, 2 * H), jnp.float32, 1)
        g_f = xgf_ref[pl.ds(rf, BC), :] + jnp.concatenate([mf0, mf1], axis=1)
        g_b = xgb_ref[pl.ds(rb, BC), :] + jnp.concatenate([mb0, mb1], axis=1)
        hf_new, cf_new = gates(g_f, c_f)
        hb_new, cb_new = gates(g_b, c_b)
        h_f, c_f = hf_new, cf_new
        out_f = jnp.where(s == cap_f_th, h_f, out_f)
        upd_b = s >= th_b
        h_b = jnp.where(upd_b, hb_new, h_b)
        c_b = jnp.where(upd_b, cb_new, c_b)
        out_b = jnp.where(s == th_b, h_b, out_b)
        return h_f, c_f, h_b, c_b, out_f, out_b

    carry = (zeros, zeros, zeros, zeros, zeros, zeros)
    for s in range(T):
        carry = body(s, carry)
    _, _, _, _, out_f, out_b = carry

    feat = jnp.concatenate([out_f, out_b], axis=1)
    w1p = jnp.concatenate(
        [w1_ref[...].astype(jnp.bfloat16),
         jnp.zeros((2 * H, 2 * H - w1_ref.shape[1]), jnp.bfloat16)],
        axis=1)
    pltpu.matmul_push_rhs(w1p, 0, 0)
    pltpu.matmul_acc_lhs(0, feat.astype(jnp.bfloat16), 0, load_staged_rhs=0)
    z1p = pltpu.matmul_pop(0, (BC, 2 * H), jnp.float32, 0)
    F = w1_ref.shape[1]
    z1 = z1p[:, 0:F] + b1_ref[...]
    w2p = jnp.concatenate(
        [w2_ref[...].astype(jnp.bfloat16),
         jnp.zeros((F, 2 * H - w2_ref.shape[1]), jnp.bfloat16)], axis=1)
    w2t = jnp.concatenate(
        [w2p, jnp.zeros((2 * H - F, 2 * H), jnp.bfloat16)], axis=0)
    pltpu.matmul_push_rhs(w2t, 0, 0)
    z1b = z1.astype(jnp.bfloat16)
    z1w = jnp.concatenate([z1b, z1b, z1b, z1b], axis=1)
    pltpu.matmul_acc_lhs(0, z1w, 0, load_staged_rhs=0)
    zp = pltpu.matmul_pop(0, (BC, 2 * H), jnp.float32, 0)
    z = zp[:, 0:out_ref.shape[1]] + b2_ref[...]
    m = jnp.max(z, axis=1, keepdims=True)
    lse = m + jnp.log(jnp.sum(jnp.exp(z - m), axis=1, keepdims=True))
    out_ref[...] = z - lse


def _bcast_spec(shape):
    nd = len(shape)
    return pl.BlockSpec(shape, lambda i, nd=nd: (0,) * nd)


def kernel(sentence, lengths, embedding, wih_f, whh_f, b_f, wih_b, whh_b,
           b_b, w1, b1, w2, b2):
    B, T = sentence.shape
    E = embedding.shape[1]
    H = whh_f.shape[0]
    tagset = w2.shape[1]
    BP = -(-B // 8) * 8

    x = jnp.take(embedding, sentence.T, axis=0)
    if BP != B:
        x = jnp.pad(x, ((0, 0), (0, BP - B), (0, 0)))
        lens_col = jnp.pad(lengths.astype(jnp.int32), (0, BP - B),
                           constant_values=1).reshape(BP, 1)
    else:
        lens_col = lengths.astype(jnp.int32).reshape(BP, 1)
    in_specs = [
        pl.BlockSpec((T, BP, E), lambda i: (0, 0, 0)),
        pl.BlockSpec((BP, 1), lambda i: (0, 0)),
        _bcast_spec(wih_f.shape),
        _bcast_spec(b_f.shape),
        _bcast_spec(whh_f.shape),
        _bcast_spec(wih_b.shape),
        _bcast_spec(b_b.shape),
        _bcast_spec(whh_b.shape),
        _bcast_spec(w1.shape),
        _bcast_spec(b1.shape),
        _bcast_spec(w2.shape),
        _bcast_spec(b2.shape),
    ]

    out = pl.pallas_call(
        _bilstm_kernel,
        out_shape=jax.ShapeDtypeStruct((BP, tagset), jnp.float32),
        grid=(1,),
        in_specs=in_specs,
        out_specs=pl.BlockSpec((BP, tagset), lambda i: (0, 0)),
        scratch_shapes=[pltpu.VMEM((T * BP, 4 * H), jnp.float32),
                        pltpu.VMEM((T * BP, 4 * H), jnp.float32)],
        compiler_params=pltpu.CompilerParams(
            dimension_semantics=("arbitrary",)),
    )(x, lens_col, wih_f, b_f, whh_f, wih_b, b_b, whh_b,
      w1, b1, w2, b2)
    return out[:B] if BP != B else out
